# baseline (speedup 1.0000x reference)
import functools

import jax
import jax.numpy as jnp
from jax.experimental import pallas as pl


def _proj_kernel(x_ref, w_ref, wl_ref, wr_ref, h_ref, el_ref, er_ref):
    x = x_ref[...]
    h_ref[...] = jnp.dot(x, w_ref[...], preferred_element_type=jnp.float32)
    el_ref[...] = jnp.dot(x, wl_ref[...], preferred_element_type=jnp.float32)
    er_ref[...] = jnp.dot(x, wr_ref[...], preferred_element_type=jnp.float32)


def _proj(x, W, al, ar, bN=1000):
    N, Fin = x.shape
    Fout = W.shape[1]
    H, D = al.shape
    Wl = (W.reshape(Fin, H, D) * al[None, :, :]).sum(-1)
    Wr = (W.reshape(Fin, H, D) * ar[None, :, :]).sum(-1)
    grid = N // bN
    h, el, er = pl.pallas_call(
        _proj_kernel,
        grid=(grid,),
        in_specs=[
            pl.BlockSpec((bN, Fin), lambda i: (i, 0)),
            pl.BlockSpec((Fin, Fout), lambda i: (0, 0)),
            pl.BlockSpec((Fin, H), lambda i: (0, 0)),
            pl.BlockSpec((Fin, H), lambda i: (0, 0)),
        ],
        out_specs=[
            pl.BlockSpec((bN, Fout), lambda i: (i, 0)),
            pl.BlockSpec((bN, H), lambda i: (i, 0)),
            pl.BlockSpec((bN, H), lambda i: (i, 0)),
        ],
        out_shape=[
            jax.ShapeDtypeStruct((N, Fout), jnp.float32),
            jax.ShapeDtypeStruct((N, H), jnp.float32),
            jax.ShapeDtypeStruct((N, H), jnp.float32),
        ],
    )(x, W, Wl, Wr)
    return h, el, er


def _gat_layer(x, src, dst, W, al, ar, b):
    N = x.shape[0]
    H, D = al.shape
    h, el, er = _proj(x, W, al, ar)
    e = jax.nn.leaky_relu(el[src] + er[dst], negative_slope=0.2)
    m = jax.ops.segment_max(e, dst, num_segments=N)
    a = jnp.exp(e - m[dst])
    denom = jax.ops.segment_sum(a, dst, num_segments=N)
    alpha = a / denom[dst]
    msg = h.reshape(N, H, D)[src] * alpha[:, :, None]
    out = jax.ops.segment_sum(msg, dst, num_segments=N)
    return out.reshape(N, H * D) + b


def kernel(x, edge_index, W1, al1, ar1, b1, W2, al2, ar2, b2, W3, al3, ar3, b3):
    src = edge_index[0]
    dst = edge_index[1]
    h = _gat_layer(x, src, dst, W1, al1, ar1, b1)
    h = jax.nn.elu(h)
    h = _gat_layer(h, src, dst, W2, al2, ar2, b2)
    h = jax.nn.elu(h)
    h = _gat_layer(h, src, dst, W3, al3, ar3, b3)
    return h


# speedup vs baseline: 6.3358x; 6.3358x over previous
import functools

import jax
import jax.numpy as jnp
from jax import lax
from jax.experimental import pallas as pl
from jax.experimental.pallas import tpu as pltpu
from jax.experimental.pallas import tpu_sc as plsc

N = 10000
E = 160000
NW = 32
EPW = 5120
BB = 64
NB = EPW // BB
NP = 10112
STRIPE = NP // 16
CW = 128
NEG_SLOPE = 0.2


def _emit_logits(i, el, er, elt_ref, ert_ref, elm_ref, erm_ref):
    bn = el.shape[0]
    zpad = jnp.zeros((bn, 128 - el.shape[1]), jnp.float32)
    live = (i * bn + lax.broadcasted_iota(jnp.int32, (bn, 1), 0)) < N
    elt_ref[...] = jnp.where(live, jnp.concatenate([el, zpad], axis=1), -1e9)
    ert_ref[...] = jnp.where(live, jnp.concatenate([er, zpad], axis=1), -1e9)

    @pl.when(i == 0)
    def _():
        elm_ref[...] = jnp.full_like(elm_ref, -jnp.inf)
        erm_ref[...] = jnp.full_like(erm_ref, -jnp.inf)

    elm_ref[...] = jnp.maximum(elm_ref[...], el.max(axis=0, keepdims=True))
    erm_ref[...] = jnp.maximum(erm_ref[...], er.max(axis=0, keepdims=True))


def _proj_body(x_ref, w_ref, wl_ref, wr_ref, *out_refs, nch):
    i = pl.program_id(0)
    h_refs = out_refs[:nch]
    elt_ref, ert_ref, elm_ref, erm_ref = out_refs[nch:]
    x = x_ref[...]
    h = jnp.dot(x, w_ref[...], preferred_element_type=jnp.float32)
    for c in range(nch):
        h_refs[c][...] = h[:, c * CW:(c + 1) * CW]
    el = jnp.dot(x, wl_ref[...], preferred_element_type=jnp.float32)
    er = jnp.dot(x, wr_ref[...], preferred_element_type=jnp.float32)
    _emit_logits(i, el, er, elt_ref, ert_ref, elm_ref, erm_ref)


def _asm_proj_body(p0_ref, p1_ref, d0_ref, d1_ref, bprev_ref,
                   w_ref, wl_ref, wr_ref, *out_refs, nch, nch_prev, h_prev):
    i = pl.program_id(0)
    h_refs = out_refs[:nch]
    elt_ref, ert_ref, elm_ref, erm_ref = out_refs[nch:]
    dsum = d0_ref[0][:, 0:16] + d1_ref[0][:, 0:16]
    dsum = jnp.where(dsum == 0.0, 1.0, dsum)
    cph = nch_prev // h_prev
    cols = []
    for c in range(nch_prev):
        head = c // cph
        pc = p0_ref[0, c] + p1_ref[0, c]
        xc = pc / dsum[:, head:head + 1] + bprev_ref[0, c * CW:(c + 1) * CW][None, :]
        cols.append(xc)
    x = jnp.concatenate(cols, axis=1)
    x = jnp.where(x > 0.0, x, jnp.exp(x) - 1.0)
    h = jnp.dot(x, w_ref[...], preferred_element_type=jnp.float32)
    for c in range(nch):
        h_refs[c][...] = h[:, c * CW:(c + 1) * CW]
    el = jnp.dot(x, wl_ref[...], preferred_element_type=jnp.float32)
    er = jnp.dot(x, wr_ref[...], preferred_element_type=jnp.float32)
    _emit_logits(i, el, er, elt_ref, ert_ref, elm_ref, erm_ref)


def _fold_weights(W, al, ar):
    Fin = W.shape[0]
    H, D = al.shape
    Wl = (W.reshape(Fin, H, D) * al[None, :, :]).sum(-1)
    Wr = (W.reshape(Fin, H, D) * ar[None, :, :]).sum(-1)
    return Wl, Wr


def _proj_outs(nch, H, bN):
    out_specs = (
        [pl.BlockSpec((bN, CW), lambda i: (i, 0)) for _ in range(nch)]
        + [pl.BlockSpec((bN, 128), lambda i: (i, 0)),
           pl.BlockSpec((bN, 128), lambda i: (i, 0)),
           pl.BlockSpec((1, H), lambda i: (0, 0)),
           pl.BlockSpec((1, H), lambda i: (0, 0))]
    )
    out_shape = (
        [jax.ShapeDtypeStruct((NP, CW), jnp.float32) for _ in range(nch)]
        + [jax.ShapeDtypeStruct((NP, 128), jnp.float32),
           jax.ShapeDtypeStruct((NP, 128), jnp.float32),
           jax.ShapeDtypeStruct((1, H), jnp.float32),
           jax.ShapeDtypeStruct((1, H), jnp.float32)]
    )
    return out_specs, out_shape


def _proj(x, W, al, ar, bN=632):
    Fin = x.shape[1]
    Fout = W.shape[1]
    H = al.shape[0]
    nch = Fout // CW
    Wl, Wr = _fold_weights(W, al, ar)
    out_specs, out_shape = _proj_outs(nch, H, bN)
    outs = pl.pallas_call(
        functools.partial(_proj_body, nch=nch),
        grid=(NP // bN,),
        in_specs=[
            pl.BlockSpec((bN, Fin), lambda i: (i, 0)),
            pl.BlockSpec((Fin, Fout), lambda i: (0, 0)),
            pl.BlockSpec((Fin, H), lambda i: (0, 0)),
            pl.BlockSpec((Fin, H), lambda i: (0, 0)),
        ],
        out_specs=out_specs,
        out_shape=out_shape,
    )(x, W, Wl, Wr)
    return outs[:nch], outs[nch], outs[nch + 1], outs[nch + 2], outs[nch + 3]


def _asm_proj(p, d, bprev, h_prev, W, al, ar, bN=632):
    nch_prev = p.shape[1]
    Fin = nch_prev * CW
    Fout = W.shape[1]
    H = al.shape[0]
    nch = Fout // CW
    Wl, Wr = _fold_weights(W, al, ar)
    out_specs, out_shape = _proj_outs(nch, H, bN)
    outs = pl.pallas_call(
        functools.partial(_asm_proj_body, nch=nch, nch_prev=nch_prev,
                          h_prev=h_prev),
        grid=(NP // bN,),
        in_specs=[
            pl.BlockSpec((1, nch_prev, bN, CW), lambda i: (0, 0, i, 0)),
            pl.BlockSpec((1, nch_prev, bN, CW), lambda i: (1, 0, i, 0)),
            pl.BlockSpec((1, bN, 128), lambda i: (0, i, 0)),
            pl.BlockSpec((1, bN, 128), lambda i: (1, i, 0)),
            pl.BlockSpec((1, Fin), lambda i: (0, 0)),
            pl.BlockSpec((Fin, Fout), lambda i: (0, 0)),
            pl.BlockSpec((Fin, H), lambda i: (0, 0)),
            pl.BlockSpec((Fin, H), lambda i: (0, 0)),
        ],
        out_specs=out_specs,
        out_shape=out_shape,
    )(p, p, d, d, bprev, W, Wl, Wr)
    return outs[:nch], outs[nch], outs[nch + 1], outs[nch + 2], outs[nch + 3]


def _final_body(p0_ref, p1_ref, d0_ref, d1_ref, b_ref, out_ref):
    dsum = d0_ref[0][:, 0:16] + d1_ref[0][:, 0:16]
    dsum = jnp.where(dsum == 0.0, 1.0, dsum)
    pc = p0_ref[0, 0] + p1_ref[0, 0]
    out_ref[...] = pc / dsum[:, 0:1] + b_ref[0][None, :]


def _final(p, d, b, bN=1000):
    return pl.pallas_call(
        _final_body,
        grid=(N // bN,),
        in_specs=[
            pl.BlockSpec((1, 1, bN, CW), lambda i: (0, 0, i, 0)),
            pl.BlockSpec((1, 1, bN, CW), lambda i: (1, 0, i, 0)),
            pl.BlockSpec((1, bN, 128), lambda i: (0, i, 0)),
            pl.BlockSpec((1, bN, 128), lambda i: (1, i, 0)),
            pl.BlockSpec((1, CW), lambda i: (0, 0)),
        ],
        out_specs=pl.BlockSpec((bN, CW), lambda i: (i, 0)),
        out_shape=jax.ShapeDtypeStruct((N, CW), jnp.float32),
    )(p, p, d, d, b.reshape(1, CW))


def _edge_call(hcs, elt, ert, src3, dst3, c32, zacc, nch, H):
    cph = nch // H
    gp = 16 // H
    ng = BB // gp
    nrow = (ng * 16 + 127) // 128

    def body(*refs):
        it = iter(refs)
        hc_refs = [next(it) for _ in range(nch)]
        elt_hbm, ert_hbm, src_hbm, dst_hbm, c_hbm, zacc_hbm = (
            next(it), next(it), next(it), next(it), next(it), next(it))
        outp, denomp = next(it), next(it)
        (src_v, dst_v, a_v, rows_v, c_v, acc_s, sem, sem2) = (
            next(it), next(it), next(it), next(it), next(it), next(it),
            next(it), next(it))

        core = lax.axis_index("c")
        sub = lax.axis_index("s")
        wid = sub * 2 + core
        st = sub * STRIPE

        pltpu.sync_copy(src_hbm.at[wid], src_v)
        pltpu.sync_copy(dst_hbm.at[wid], dst_v)
        pltpu.sync_copy(c_hbm, c_v)
        pltpu.sync_copy(zacc_hbm.at[pl.ds(st, STRIPE)],
                        acc_s.at[pl.ds(st, STRIPE)])

        cvec = c_v[pl.ds(0, 16)]
        mask = c_v[pl.ds(16, 16)]
        iota = lax.iota(jnp.int32, 16)

        gdn = lax.GatherDimensionNumbers(
            offset_dims=(), collapsed_slice_dims=(0,), start_index_map=(0,))

        def rot(v, k):
            idx = ((iota - k) % 16).reshape(16, 1)
            return lax.gather(v, idx, gdn, (1,),
                              mode=lax.GatherScatterMode.PROMISE_IN_BOUNDS)

        def p1(b, carry):
            pltpu.async_copy(elt_hbm.at[src_v.at[b]], rows_v, sem).wait()
            pltpu.async_copy(ert_hbm.at[dst_v.at[b]], rows_v, sem2,
                             add=True).wait()
            for g in range(ng):
                acc16 = None
                for m in range(gp):
                    e = g * gp + m
                    z = rows_v[e, pl.ds(0, 16)]
                    lr = jnp.maximum(z, NEG_SLOPE * z)
                    a16 = jnp.exp(lr - cvec)
                    r = a16 if m == 0 else rot(a16, m * H)
                    acc16 = r if acc16 is None else acc16 + r
                a_v[b, g // 8, pl.ds((g % 8) * 16, 16)] = acc16
            return carry

        lax.fori_loop(0, NB, p1, 0, unroll=False)
        plsc.subcore_barrier()

        for ci in range(nch):
            head = ci // cph

            def p2(b, carry, ci=ci, head=head):
                pltpu.async_copy(hc_refs[ci].at[src_v.at[b]], rows_v,
                                 sem).wait()
                for g in range(ng):
                    pk = a_v[b, g // 8, pl.ds((g % 8) * 16, 16)]
                    for m in range(gp):
                        e = g * gp + m
                        a_s = pk[m * H + head]
                        for j in range(CW // 16):
                            sl = pl.ds(j * 16, 16)
                            rows_v[e, sl] = rows_v[e, sl] * a_s
                pltpu.sync_copy(rows_v, acc_s.at[dst_v.at[b]], add=True)
                return carry

            lax.fori_loop(0, NB, p2, 0, unroll=False)
            plsc.subcore_barrier()
            pltpu.sync_copy(acc_s.at[pl.ds(st, STRIPE)],
                            outp.at[core, ci, pl.ds(st, STRIPE)])
            pltpu.sync_copy(zacc_hbm.at[pl.ds(st, STRIPE)],
                            acc_s.at[pl.ds(st, STRIPE)])
            plsc.subcore_barrier()

        pltpu.sync_copy(zacc_hbm.at[pl.ds(0, BB)], rows_v)

        def pden(b, carry):
            for g in range(ng):
                pk = a_v[b, g // 8, pl.ds((g % 8) * 16, 16)]
                for m in range(gp):
                    e = g * gp + m
                    row16 = rot(pk, (16 - m * H) % 16) * mask
                    rows_v[e, pl.ds(0, 16)] = row16
            pltpu.sync_copy(rows_v, acc_s.at[dst_v.at[b]], add=True)
            return carry

        lax.fori_loop(0, NB, pden, 0, unroll=False)
        plsc.subcore_barrier()
        pltpu.sync_copy(acc_s.at[pl.ds(st, STRIPE)],
                        denomp.at[core, pl.ds(st, STRIPE)])

    mesh = plsc.VectorSubcoreMesh(core_axis_name="c", subcore_axis_name="s")
    fn = pl.kernel(
        body,
        out_type=[
            jax.ShapeDtypeStruct((2, nch, NP, CW), jnp.float32),
            jax.ShapeDtypeStruct((2, NP, 128), jnp.float32),
        ],
        mesh=mesh,
        scratch_types=[
            pltpu.VMEM((NB, BB), jnp.int32),
            pltpu.VMEM((NB, BB), jnp.int32),
            pltpu.VMEM((NB, nrow, 128), jnp.float32),
            pltpu.VMEM((BB, CW), jnp.float32),
            pltpu.VMEM((32,), jnp.float32),
            pltpu.VMEM_SHARED((NP, CW), jnp.float32),
            pltpu.SemaphoreType.DMA,
            pltpu.SemaphoreType.DMA,
        ],
    )
    return fn(*hcs, elt, ert, src3, dst3, c32, zacc)


def kernel(x, edge_index, W1, al1, ar1, b1, W2, al2, ar2, b2, W3, al3, ar3, b3):
    src = edge_index[0].astype(jnp.int32)
    dst = edge_index[1].astype(jnp.int32)
    pad = NW * EPW - E
    src3 = jnp.pad(src, (0, pad), constant_values=N).reshape(NW, NB, BB)
    dst3 = jnp.pad(dst, (0, pad), constant_values=N).reshape(NW, NB, BB)
    x = jnp.pad(x, ((0, NP - N), (0, 0)))
    zacc = jnp.zeros((NP, CW), jnp.float32)

    def edge_phase(hcs, elt, ert, elm, erm, H):
        nch = len(hcs)
        cvals = jnp.maximum(elm[0] + erm[0], 0.0)
        c32 = jnp.full((32,), 1e9, jnp.float32).at[:H].set(cvals)
        c32 = c32.at[16:16 + H].set(1.0).at[16 + H:].set(0.0)
        return _edge_call(hcs, elt, ert, src3, dst3, c32, zacc, nch, H)

    hcs, elt, ert, elm, erm = _proj(x, W1, al1, ar1)
    p, d = edge_phase(hcs, elt, ert, elm, erm, 4)
    hcs, elt, ert, elm, erm = _asm_proj(p, d, b1.reshape(1, -1), 4, W2, al2, ar2)
    p, d = edge_phase(hcs, elt, ert, elm, erm, 4)
    hcs, elt, ert, elm, erm = _asm_proj(p, d, b2.reshape(1, -1), 4, W3, al3, ar3)
    p, d = edge_phase(hcs, elt, ert, elm, erm, 1)
    return _final(p, d, b3)
